# baseline (speedup 1.0000x reference)
import jax
import jax.numpy as jnp
from jax import lax
from jax.experimental import pallas as pl
from jax.experimental.pallas import tpu as pltpu
from jax.experimental.pallas import tpu_sc as plsc

ROWS = 16
COLS = 20000
NC = 2
NS = 16
L = 16
SHARP = 10.0
NEG = float("-inf")

W = 640
TAILC0 = 31 * W
TAILW = COLS - TAILC0
XW = 128 + W + 128


def _body(x_hbm, lg_hbm, sp_hbm, mk_hbm, pv_hbm,
          xb0, xb1, sp0, sp1, mk0, mk1, pv0, pv1, lgb, sem):
    cid = lax.axis_index("c")
    sid = lax.axis_index("s")
    wid = sid * NC + cid

    pltpu.sync_copy(lg_hbm, lgb)
    logit = lgb[...]
    thresh = 1.0 / (1.0 + jnp.exp(-logit))
    neg = jnp.full((L,), NEG, jnp.float32)

    xbs = (xb0, xb1)
    outs = ((sp0, mk0, pv0), (sp1, mk1, pv1))

    def compute(xb, spb, mkb, pvb, width):
        nv = width // L

        @plsc.parallel_loop(0, 8, step=1, unroll=1)
        def row(rr):
            for k in range(nv):
                col = 128 + k * L
                x = xb[rr, pl.ds(col, L)]
                a = jnp.maximum(xb[rr, pl.ds(col - 2, L)], xb[rr, pl.ds(col - 1, L)])
                b = jnp.maximum(xb[rr, pl.ds(col + 1, L)], xb[rr, pl.ds(col + 2, L)])
                pooled = jnp.maximum(x, jnp.maximum(a, b))
                ea = jnp.exp(SHARP * (thresh - x))
                eb = jnp.exp(SHARP * (pooled - x))
                sp = x / (1.0 + ea + eb + ea * eb)
                keep = sp >= thresh
                o = k * L
                spb[rr, pl.ds(o, L)] = sp
                mkb[rr, pl.ds(o, L)] = jnp.where(keep, 1.0, 0.0).astype(jnp.float32)
                pvb[rr, pl.ds(o, L)] = jnp.where(keep, x, 0.0)

    @pl.when(wid < 31)
    def _main():
        c0 = wid * W
        cl = pl.multiple_of(jnp.maximum(c0 - 128, 0), 128)
        cps = []
        for g in range(2):
            xb = xbs[g]
            r0 = 8 * g
            cps.append(pltpu.async_copy(
                x_hbm.at[pl.ds(r0, 8), pl.ds(cl, 128)], xb.at[:, pl.ds(0, 128)], sem))
            for j in range(5):
                cps.append(pltpu.async_copy(
                    x_hbm.at[pl.ds(r0, 8), pl.ds(pl.multiple_of(c0 + 128 * j, 128), 128)],
                    xb.at[:, pl.ds(128 + 128 * j, 128)], sem))
            cps.append(pltpu.async_copy(
                x_hbm.at[pl.ds(r0, 8), pl.ds(pl.multiple_of(c0 + W, 128), 128)],
                xb.at[:, pl.ds(128 + W, 128)], sem))
        for cp in cps:
            cp.wait()

        @pl.when(wid == 0)
        def _padleft():
            for g in range(2):
                for rr in range(8):
                    xbs[g][rr, pl.ds(112, L)] = neg

        for g in range(2):
            compute(xbs[g], *outs[g], W)

        cps = []
        for g in range(2):
            r0 = 8 * g
            for (buf, hbm) in zip(outs[g], (sp_hbm, mk_hbm, pv_hbm)):
                for j in range(5):
                    cps.append(pltpu.async_copy(
                        buf.at[:, pl.ds(128 * j, 128)],
                        hbm.at[pl.ds(r0, 8), pl.ds(pl.multiple_of(c0 + 128 * j, 128), 128)],
                        sem))
        for cp in cps:
            cp.wait()

    @pl.when(wid == 31)
    def _tail():
        tpad = pl.multiple_of(TAILC0 + 128 + cid * 0, 128)
        cps = []
        for g in range(2):
            xb = xbs[g]
            r0 = 8 * g
            cps.append(pltpu.async_copy(
                x_hbm.at[pl.ds(r0, 8), pl.ds(TAILC0 - 128, 128)],
                xb.at[:, pl.ds(0, 128)], sem))
            cps.append(pltpu.async_copy(
                x_hbm.at[pl.ds(r0, 8), pl.ds(TAILC0, 128)],
                xb.at[:, pl.ds(128, 128)], sem))
            cps.append(pltpu.async_copy(
                x_hbm.at[pl.ds(r0, 8), pl.ds(tpad, 128)],
                xb.at[:, pl.ds(256, 128)], sem))
        for cp in cps:
            cp.wait()
        for g in range(2):
            for rr in range(8):
                xbs[g][rr, pl.ds(128 + TAILW, L)] = neg

        for g in range(2):
            compute(xbs[g], *outs[g], TAILW)

        cps = []
        for g in range(2):
            r0 = 8 * g
            for (buf, hbm) in zip(outs[g], (sp_hbm, mk_hbm, pv_hbm)):
                cps.append(pltpu.async_copy(
                    buf.at[:, pl.ds(0, 128)],
                    hbm.at[pl.ds(r0, 8), pl.ds(TAILC0, 128)], sem))
                cps.append(pltpu.async_copy(
                    buf.at[:, pl.ds(128, 128)],
                    hbm.at[pl.ds(r0, 8), pl.ds(tpad, 128)], sem))
        for cp in cps:
            cp.wait()


@jax.jit
def _run(peak_map, logit_vec):
    mesh = plsc.VectorSubcoreMesh(
        core_axis_name="c", subcore_axis_name="s", num_cores=NC, num_subcores=NS
    )
    f = pl.kernel(
        _body,
        out_type=(
            jax.ShapeDtypeStruct((ROWS, COLS), jnp.float32),
            jax.ShapeDtypeStruct((ROWS, COLS), jnp.float32),
            jax.ShapeDtypeStruct((ROWS, COLS), jnp.float32),
        ),
        mesh=mesh,
        scratch_types=[
            pltpu.VMEM((8, XW), jnp.float32),
            pltpu.VMEM((8, XW), jnp.float32),
        ] + [pltpu.VMEM((8, W), jnp.float32) for _ in range(6)] + [
            pltpu.VMEM((L,), jnp.float32),
            pltpu.SemaphoreType.DMA,
        ],
    )
    return f(peak_map, logit_vec)


def kernel(peak_map, logit_thresh):
    logit_vec = jnp.broadcast_to(logit_thresh.astype(jnp.float32), (L,))
    sp, mk, pv = _run(peak_map, logit_vec)
    return sp, mk != 0.0, pv


# speedup vs baseline: 1.4215x; 1.4215x over previous
import jax
import jax.numpy as jnp
from jax import lax
from jax.experimental import pallas as pl
from jax.experimental.pallas import tpu as pltpu
from jax.experimental.pallas import tpu_sc as plsc

ROWS = 16
COLS = 20000
NC = 2
NS = 16
L = 16
SHARP = 10.0
NEG = float("-inf")

W = 640
NT = W // 128
TAILC0 = 31 * W
TAILW = COLS - TAILC0
XW = 128 + W + 128


def _body(x_hbm, lg_hbm, sp_hbm, mk_hbm, pv_hbm,
          xb2, xb1, spb, mkb, pvb, lgb, sem, sem2):
    cid = lax.axis_index("c")
    sid = lax.axis_index("s")
    wid = sid * NC + cid

    pltpu.sync_copy(lg_hbm, lgb)
    logit = lgb[...]
    thresh = 1.0 / (1.0 + jnp.exp(-logit))
    neg = jnp.full((L,), NEG, jnp.float32)

    def repack():
        nm = XW // L

        @plsc.parallel_loop(0, ROWS * nm, step=1, unroll=8)
        def mv(v):
            r = v // nm
            m = v - r * nm
            xb1[pl.ds(r * XW + m * L, L)] = xb2[r, pl.ds(m * L, L)]

    def compute(width):
        nv = width // L

        @plsc.parallel_loop(0, ROWS * nv, step=1, unroll=4)
        def step(v):
            r = v // nv
            k = v - r * nv
            base = r * XW + 128 + k * L
            x = xb1[pl.ds(base, L)]
            a = jnp.maximum(xb1[pl.ds(base - 2, L)], xb1[pl.ds(base - 1, L)])
            b = jnp.maximum(xb1[pl.ds(base + 1, L)], xb1[pl.ds(base + 2, L)])
            pooled = jnp.maximum(x, jnp.maximum(a, b))
            ea = jnp.exp(SHARP * (thresh - x))
            eb = jnp.exp(SHARP * (pooled - x))
            sp = x / (1.0 + ea + eb + ea * eb)
            keep = sp >= thresh
            o = k * L
            spb[r, pl.ds(o, L)] = sp
            mkb[r, pl.ds(o, L)] = jnp.where(keep, 1.0, 0.0).astype(jnp.float32)
            pvb[r, pl.ds(o, L)] = jnp.where(keep, x, 0.0)

    def store_tiles(c0, ntiles):
        cps = []
        for g in range(2):
            r0 = 8 * g
            for (buf, hbm) in zip((spb, mkb, pvb), (sp_hbm, mk_hbm, pv_hbm)):
                for j in range(ntiles):
                    cps.append(pltpu.async_copy(
                        buf.at[pl.ds(r0, 8), pl.ds(128 * j, 128)],
                        hbm.at[pl.ds(r0, 8),
                               pl.ds(pl.multiple_of(c0 + 128 * j, 128), 128)],
                        sem))
        for cp in cps:
            cp.wait()

    @pl.when(wid < 31)
    def _main():
        c0 = wid * W
        cl = pl.multiple_of(jnp.maximum(c0 - 128, 0), 128)
        cps = []
        for g in range(2):
            r0 = 8 * g
            cps.append(pltpu.async_copy(
                x_hbm.at[pl.ds(r0, 8), pl.ds(cl, 128)],
                xb2.at[pl.ds(r0, 8), pl.ds(0, 128)], sem))
            for j in range(NT + 1):
                cps.append(pltpu.async_copy(
                    x_hbm.at[pl.ds(r0, 8),
                             pl.ds(pl.multiple_of(c0 + 128 * j, 128), 128)],
                    xb2.at[pl.ds(r0, 8), pl.ds(128 + 128 * j, 128)], sem))
        for cp in cps:
            cp.wait()
        repack()

        @pl.when(wid == 0)
        def _padleft():
            for r in range(ROWS):
                xb1[pl.ds(r * XW + 112, L)] = neg

        compute(W)
        store_tiles(c0, NT)

    @pl.when(wid == 31)
    def _tail():
        tpad = pl.multiple_of(TAILC0 + 128 + cid * 0, 128)
        cps = []
        for g in range(2):
            r0 = 8 * g
            cps.append(pltpu.async_copy(
                x_hbm.at[pl.ds(r0, 8), pl.ds(TAILC0 - 128, 128)],
                xb2.at[pl.ds(r0, 8), pl.ds(0, 128)], sem))
            cps.append(pltpu.async_copy(
                x_hbm.at[pl.ds(r0, 8), pl.ds(TAILC0, 128)],
                xb2.at[pl.ds(r0, 8), pl.ds(128, 128)], sem))
            cps.append(pltpu.async_copy(
                x_hbm.at[pl.ds(r0, 8), pl.ds(tpad, 128)],
                xb2.at[pl.ds(r0, 8), pl.ds(256, 128)], sem))
        for cp in cps:
            cp.wait()
        repack()
        for r in range(ROWS):
            xb1[pl.ds(r * XW + 128 + TAILW, L)] = neg

        compute(TAILW)

        cps = []
        for g in range(2):
            r0 = 8 * g
            for (buf, hbm) in zip((spb, mkb, pvb), (sp_hbm, mk_hbm, pv_hbm)):
                cps.append(pltpu.async_copy(
                    buf.at[pl.ds(r0, 8), pl.ds(0, 128)],
                    hbm.at[pl.ds(r0, 8), pl.ds(TAILC0, 128)], sem))
                cps.append(pltpu.async_copy(
                    buf.at[pl.ds(r0, 8), pl.ds(128, 128)],
                    hbm.at[pl.ds(r0, 8), pl.ds(tpad, 128)], sem))
        for cp in cps:
            cp.wait()


@jax.jit
def _run(peak_map, logit_vec):
    mesh = plsc.VectorSubcoreMesh(
        core_axis_name="c", subcore_axis_name="s", num_cores=NC, num_subcores=NS
    )
    f = pl.kernel(
        _body,
        out_type=(
            jax.ShapeDtypeStruct((ROWS, COLS), jnp.float32),
            jax.ShapeDtypeStruct((ROWS, COLS), jnp.float32),
            jax.ShapeDtypeStruct((ROWS, COLS), jnp.float32),
        ),
        mesh=mesh,
        scratch_types=[
            pltpu.VMEM((ROWS, XW), jnp.float32),
            pltpu.VMEM((ROWS * XW,), jnp.float32),
            pltpu.VMEM((ROWS, W), jnp.float32),
            pltpu.VMEM((ROWS, W), jnp.float32),
            pltpu.VMEM((ROWS, W), jnp.float32),
            pltpu.VMEM((L,), jnp.float32),
            pltpu.SemaphoreType.DMA,
            pltpu.SemaphoreType.DMA,
        ],
    )
    return f(peak_map, logit_vec)


def kernel(peak_map, logit_thresh):
    logit_vec = jnp.broadcast_to(logit_thresh.astype(jnp.float32), (L,))
    sp, mk, pv = _run(peak_map, logit_vec)
    return sp, mk != 0.0, pv


# speedup vs baseline: 1.4628x; 1.0290x over previous
import jax
import jax.numpy as jnp
from jax import lax
from jax.experimental import pallas as pl
from jax.experimental.pallas import tpu as pltpu
from jax.experimental.pallas import tpu_sc as plsc

ROWS = 16
COLS = 20000
NC = 2
NS = 16
L = 16
SHARP = 10.0
NEG = float("-inf")

W = 640
COLS_PAD = 157 * 128
LAST_TILE = 156 * 128
C0_MAX = COLS_PAD - W
XW = 128 + W + 128
NV = W // L


def _body(x_hbm, lg_hbm, sp_hbm, mk_hbm, pv_hbm,
          xb2, xb1, spb, mkb, pvb, lgb, sem):
    cid = lax.axis_index("c")
    sid = lax.axis_index("s")
    wid = sid * NC + cid

    pltpu.sync_copy(lg_hbm, lgb)
    logit = lgb[...]
    thresh = 1.0 / (1.0 + jnp.exp(-logit))
    neg = jnp.full((L,), NEG, jnp.float32)

    c0 = pl.multiple_of(jnp.minimum(wid * W, C0_MAX), 128)
    cl = pl.multiple_of(jnp.maximum(c0 - 128, 0), 128)
    cr = pl.multiple_of(jnp.minimum(c0 + W, LAST_TILE), 128)

    cps = []
    for g in range(2):
        r0 = 8 * g
        cps.append(pltpu.async_copy(
            x_hbm.at[pl.ds(r0, 8), pl.ds(cl, 128)],
            xb2.at[pl.ds(r0, 8), pl.ds(0, 128)], sem))
        cps.append(pltpu.async_copy(
            x_hbm.at[pl.ds(r0, 8), pl.ds(c0, W)],
            xb2.at[pl.ds(r0, 8), pl.ds(128, W)], sem))
        cps.append(pltpu.async_copy(
            x_hbm.at[pl.ds(r0, 8), pl.ds(cr, 128)],
            xb2.at[pl.ds(r0, 8), pl.ds(128 + W, 128)], sem))
    for cp in cps:
        cp.wait()

    NM = XW // L

    @plsc.parallel_loop(0, ROWS * NM, step=1, unroll=8)
    def mv(v):
        r = v // NM
        m = v - r * NM
        xb1[pl.ds(r * XW + m * L, L)] = xb2[r, pl.ds(m * L, L)]

    @pl.when(wid == 0)
    def _padleft():
        for r in range(ROWS):
            xb1[pl.ds(r * XW + 112, L)] = neg

    @pl.when(wid == 31)
    def _padright():
        for r in range(ROWS):
            xb1[pl.ds(r * XW + 128 + (COLS - C0_MAX), L)] = neg

    @plsc.parallel_loop(0, ROWS * NV, step=1, unroll=4)
    def step(v):
        r = v // NV
        k = v - r * NV
        base = r * XW + 128 + k * L
        x = xb1[pl.ds(base, L)]
        a = jnp.maximum(xb1[pl.ds(base - 2, L)], xb1[pl.ds(base - 1, L)])
        b = jnp.maximum(xb1[pl.ds(base + 1, L)], xb1[pl.ds(base + 2, L)])
        pooled = jnp.maximum(x, jnp.maximum(a, b))
        ea = jnp.exp(SHARP * (thresh - x))
        eb = jnp.exp(SHARP * (pooled - x))
        sp = x / (1.0 + ea + eb + ea * eb)
        keep = sp >= thresh
        o = k * L
        spb[r, pl.ds(o, L)] = sp
        mkb[r, pl.ds(o, L)] = jnp.where(keep, 1.0, 0.0).astype(jnp.float32)
        pvb[r, pl.ds(o, L)] = jnp.where(keep, x, 0.0)

    cps = []
    for g in range(2):
        r0 = 8 * g
        for (buf, hbm) in zip((spb, mkb, pvb), (sp_hbm, mk_hbm, pv_hbm)):
            cps.append(pltpu.async_copy(
                buf.at[pl.ds(r0, 8), pl.ds(0, W)],
                hbm.at[pl.ds(r0, 8), pl.ds(c0, W)], sem))
    for cp in cps:
        cp.wait()


@jax.jit
def _run(peak_map, logit_vec):
    mesh = plsc.VectorSubcoreMesh(
        core_axis_name="c", subcore_axis_name="s", num_cores=NC, num_subcores=NS
    )
    f = pl.kernel(
        _body,
        out_type=(
            jax.ShapeDtypeStruct((ROWS, COLS), jnp.float32),
            jax.ShapeDtypeStruct((ROWS, COLS), jnp.float32),
            jax.ShapeDtypeStruct((ROWS, COLS), jnp.float32),
        ),
        mesh=mesh,
        scratch_types=[
            pltpu.VMEM((ROWS, XW), jnp.float32),
            pltpu.VMEM((ROWS * XW,), jnp.float32),
            pltpu.VMEM((ROWS, W), jnp.float32),
            pltpu.VMEM((ROWS, W), jnp.float32),
            pltpu.VMEM((ROWS, W), jnp.float32),
            pltpu.VMEM((L,), jnp.float32),
            pltpu.SemaphoreType.DMA,
        ],
    )
    return f(peak_map, logit_vec)


def kernel(peak_map, logit_thresh):
    logit_vec = jnp.broadcast_to(logit_thresh.astype(jnp.float32), (L,))
    sp, mk, pv = _run(peak_map, logit_vec)
    return sp, mk != 0.0, pv


# speedup vs baseline: 1.5241x; 1.0419x over previous
import jax
import jax.numpy as jnp
from jax import lax
from jax.experimental import pallas as pl
from jax.experimental.pallas import tpu as pltpu
from jax.experimental.pallas import tpu_sc as plsc

ROWS = 16
COLS = 20000
NC = 2
NS = 16
L = 16
SHARP = 10.0
NEG = float("-inf")

W = 640
COLS_PAD = 157 * 128
LAST_TILE = 156 * 128
C0_MAX = COLS_PAD - W
XW = 128 + W + 128
NV = W // L


def _body(x_hbm, lg_hbm, sp_hbm, pv_hbm,
          xb2, xb1, spb, pvb, lgb, sem):
    cid = lax.axis_index("c")
    sid = lax.axis_index("s")
    wid = sid * NC + cid

    pltpu.sync_copy(lg_hbm, lgb)
    logit = lgb[...]
    thresh = 1.0 / (1.0 + jnp.exp(-logit))
    t10 = SHARP * thresh
    neg = jnp.full((L,), NEG, jnp.float32)

    c0 = pl.multiple_of(jnp.minimum(wid * W, C0_MAX), 128)
    cl = pl.multiple_of(jnp.maximum(c0 - 128, 0), 128)
    cr = pl.multiple_of(jnp.minimum(c0 + W, LAST_TILE), 128)

    cps = []
    for g in range(2):
        r0 = 8 * g
        cps.append(pltpu.async_copy(
            x_hbm.at[pl.ds(r0, 8), pl.ds(cl, 128)],
            xb2.at[pl.ds(r0, 8), pl.ds(0, 128)], sem))
        cps.append(pltpu.async_copy(
            x_hbm.at[pl.ds(r0, 8), pl.ds(c0, W)],
            xb2.at[pl.ds(r0, 8), pl.ds(128, W)], sem))
        cps.append(pltpu.async_copy(
            x_hbm.at[pl.ds(r0, 8), pl.ds(cr, 128)],
            xb2.at[pl.ds(r0, 8), pl.ds(128 + W, 128)], sem))

    NM = XW // L
    ocps = []
    for g in range(2):
        r0 = 8 * g
        for cp in cps[3 * g:3 * g + 3]:
            cp.wait()

        @plsc.parallel_loop(0, 8 * NM, step=1, unroll=8)
        def mv(v):
            r = v // NM
            m = v - r * NM
            xb1[pl.ds((r0 + r) * XW + m * L, L)] = xb2[r0 + r, pl.ds(m * L, L)]

        @pl.when(wid == 0)
        def _padleft():
            for r in range(8):
                xb1[pl.ds((r0 + r) * XW + 112, L)] = neg

        @pl.when(wid == 31)
        def _padright():
            for r in range(8):
                xb1[pl.ds((r0 + r) * XW + 128 + (COLS - C0_MAX), L)] = neg

        @plsc.parallel_loop(0, 8 * NV, step=1, unroll=4)
        def step(v):
            r = v // NV
            k = v - r * NV
            base = (r0 + r) * XW + 128 + k * L
            x = xb1[pl.ds(base, L)]
            a = jnp.maximum(xb1[pl.ds(base - 2, L)], xb1[pl.ds(base - 1, L)])
            b = jnp.maximum(xb1[pl.ds(base + 1, L)], xb1[pl.ds(base + 2, L)])
            pooled = jnp.maximum(x, jnp.maximum(a, b))
            x10 = SHARP * x
            ea = jnp.exp(t10 - x10)
            eb = jnp.exp(SHARP * pooled - x10)
            sp = x / ((1.0 + ea) * (1.0 + eb))
            o = k * L
            spb[r0 + r, pl.ds(o, L)] = sp
            pvb[r0 + r, pl.ds(o, L)] = jnp.where(sp >= thresh, x, 0.0)

        for (buf, hbm) in ((spb, sp_hbm), (pvb, pv_hbm)):
            ocps.append(pltpu.async_copy(
                buf.at[pl.ds(r0, 8), pl.ds(0, W)],
                hbm.at[pl.ds(r0, 8), pl.ds(c0, W)], sem))
    for cp in ocps:
        cp.wait()


@jax.jit
def _run(peak_map, logit_vec):
    mesh = plsc.VectorSubcoreMesh(
        core_axis_name="c", subcore_axis_name="s", num_cores=NC, num_subcores=NS
    )
    f = pl.kernel(
        _body,
        out_type=(
            jax.ShapeDtypeStruct((ROWS, COLS), jnp.float32),
            jax.ShapeDtypeStruct((ROWS, COLS), jnp.float32),
        ),
        mesh=mesh,
        scratch_types=[
            pltpu.VMEM((ROWS, XW), jnp.float32),
            pltpu.VMEM((ROWS * XW,), jnp.float32),
            pltpu.VMEM((ROWS, W), jnp.float32),
            pltpu.VMEM((ROWS, W), jnp.float32),
            pltpu.VMEM((L,), jnp.float32),
            pltpu.SemaphoreType.DMA,
        ],
    )
    return f(peak_map, logit_vec)


def kernel(peak_map, logit_thresh):
    logit_vec = jnp.broadcast_to(logit_thresh.astype(jnp.float32), (L,))
    sp, pv = _run(peak_map, logit_vec)
    return sp, sp >= jax.nn.sigmoid(logit_thresh), pv
